# baseline (speedup 1.0000x reference)
import functools
import math

import jax
import jax.numpy as jnp
from jax import lax
from jax.experimental import pallas as pl
from jax.experimental.pallas import tpu as pltpu
from jax.experimental.pallas import tpu_sc as plsc

H = 128
H3 = 3 * H
TW = 2 * H3
TWP = TW // 2
HP = H3 // 2
NRAD = 20

NC = 2
NS = 16
NW = NC * NS

CH = 80
GRP = 8


def _pack_bf16(x):
    b = lax.bitcast_convert_type(x, jnp.uint32)
    rnd = b + jnp.uint32(0x7FFF) + (lax.shift_right_logical(b, jnp.uint32(16)) & jnp.uint32(1))
    lo = lax.shift_right_logical(rnd[:, :HP], jnp.uint32(16))
    hi = rnd[:, HP:] & jnp.uint32(0xFFFF0000)
    return lax.bitcast_convert_type(lo | hi, jnp.int32)


def _unpack_bf16(p):
    u = lax.bitcast_convert_type(p, jnp.uint32)
    lo = lax.shift_left(u, jnp.uint32(16))
    hi = u & jnp.uint32(0xFFFF0000)
    return jnp.concatenate(
        [lax.bitcast_convert_type(lo, jnp.float32),
         lax.bitcast_convert_type(hi, jnp.float32)], axis=1)


def _mlp_body(s_ref, vj_ref, w1_ref, b1_ref, w2_ref, b2_ref, out_ref):
    s = s_ref[...]
    h = lax.dot_general(s, w1_ref[...], (((1,), (1,)), ((), ())),
                        preferred_element_type=jnp.float32) + b1_ref[...]
    h = h * jax.nn.sigmoid(h)
    mf = lax.dot_general(h, w2_ref[...], (((1,), (1,)), ((), ())),
                         preferred_element_type=jnp.float32) + b2_ref[...]
    out_ref[:, :HP] = _pack_bf16(mf)
    out_ref[:, HP:] = _pack_bf16(vj_ref[...])


def _node_table(s_j, vj2, W1, b1, W2, b2):
    n = s_j.shape[0]
    bn = 1000
    return pl.pallas_call(
        _mlp_body,
        grid=(n // bn,),
        in_specs=[
            pl.BlockSpec((bn, H), lambda i: (i, 0)),
            pl.BlockSpec((bn, H3), lambda i: (i, 0)),
            pl.BlockSpec((H, H), lambda i: (0, 0)),
            pl.BlockSpec((1, H), lambda i: (0, 0)),
            pl.BlockSpec((H3, H), lambda i: (0, 0)),
            pl.BlockSpec((1, H3), lambda i: (0, 0)),
        ],
        out_specs=pl.BlockSpec((bn, TWP), lambda i: (i, 0)),
        out_shape=jax.ShapeDtypeStruct((n, TWP), jnp.int32),
        compiler_params=pltpu.CompilerParams(
            dimension_semantics=("parallel",)),
    )(s_j, vj2, W1, b1.reshape(1, H), W2, b2.reshape(1, H3))


def _make_gather(n, e):
    nunits = e // (CH * GRP)
    iters = -(-nunits // NW)
    mesh = plsc.VectorSubcoreMesh(core_axis_name="c", subcore_axis_name="s")

    @functools.partial(
        pl.kernel,
        out_type=jax.ShapeDtypeStruct((e, TWP), jnp.int32),
        mesh=mesh,
        scratch_types=(
            [pltpu.VMEM((GRP, CH), jnp.int32)]
            + [pltpu.VMEM((CH, TWP), jnp.int32) for _ in range(4)]
            + [pltpu.SemaphoreType.DMA for _ in range(8)]
        ),
    )
    def gather(tab_hbm, src2_hbm, g_hbm, idxg, b0, b1, b2, b3,
               g0, g1, g2, g3, w0, w1, w2, w3):
        wid = lax.axis_index("s") * NC + lax.axis_index("c")
        bufs = (b0, b1, b2, b3)
        gsems = (g0, g1, g2, g3)
        wsems = (w0, w1, w2, w3)

        @pl.loop(0, iters)
        def _(i):
            unit = i * NW + wid

            @pl.when(unit < nunits)
            def _():
                row0 = unit * GRP
                pltpu.sync_copy(src2_hbm.at[pl.ds(row0, GRP)], idxg)
                gh = [None] * GRP
                wh = [None] * GRP
                for k in range(GRP + 1):
                    if k < GRP:
                        b = k % 4
                        if k >= 4:
                            wh[k - 4].wait()
                        gh[k] = pltpu.async_copy(
                            tab_hbm.at[idxg.at[k]], bufs[b], gsems[b])
                    if k >= 1:
                        j = k - 1
                        gh[j].wait()
                        wh[j] = pltpu.async_copy(
                            bufs[j % 4],
                            g_hbm.at[pl.ds((row0 + j) * CH, CH)],
                            wsems[j % 4])
                for j in range(GRP - 4, GRP):
                    wh[j].wait()

    return gather


_S_VS = 1.0 / (math.sqrt(3.0) * math.sqrt(float(H)))
_S_VV = 1.0 / math.sqrt(float(H))


def _edge_body(g_ref, rbf_ref, vec_ref, wr_ref, br_ref, w_ref):
    wf = lax.dot_general(rbf_ref[...], wr_ref[...], (((1,), (1,)), ((), ())),
                         preferred_element_type=jnp.float32) + br_ref[...]
    gi = g_ref[...]
    mf_j = _unpack_bf16(gi[:, :HP])
    v_j_j = _unpack_bf16(gi[:, HP:])
    prod = mf_j * wf
    w_ref[0] = prod[:, :H]
    w_vs = prod[:, H:2 * H] * _S_VS
    w_vv = prod[:, 2 * H:H3] * _S_VV
    vec = vec_ref[...]
    for c in range(3):
        w_ref[c + 1] = v_j_j[:, c * H:(c + 1) * H] * w_vs + w_vv * vec[:, c:c + 1]


def _edge_math(g, edge_rbf, edge_vec, Wr, br):
    e = g.shape[0]
    be = 2000
    return pl.pallas_call(
        _edge_body,
        grid=(e // be,),
        in_specs=[
            pl.BlockSpec((be, TWP), lambda i: (i, 0)),
            pl.BlockSpec((be, NRAD), lambda i: (i, 0)),
            pl.BlockSpec((be, 3), lambda i: (i, 0)),
            pl.BlockSpec((H3, NRAD), lambda i: (0, 0)),
            pl.BlockSpec((1, H3), lambda i: (0, 0)),
        ],
        out_specs=pl.BlockSpec((4, be, H), lambda i: (0, i, 0)),
        out_shape=jax.ShapeDtypeStruct((4, e, H), jnp.float32),
        compiler_params=pltpu.CompilerParams(
            dimension_semantics=("parallel",)),
    )(g, edge_rbf, edge_vec, Wr, br.reshape(1, H3))


def _make_scatter(n, e, nslab):
    es = e // nslab
    srows = es // CH
    nunits = es // (CH * GRP)
    iters = -(-nunits // NS)
    nwb = 10
    nrow = n // nwb
    mesh = plsc.VectorSubcoreMesh(core_axis_name="c", subcore_axis_name="s")

    @functools.partial(
        pl.kernel,
        out_type=jax.ShapeDtypeStruct((4, n, H), jnp.float32),
        mesh=mesh,
        scratch_types=(
            [pltpu.VMEM((GRP, CH), jnp.int32)]
            + [pltpu.VMEM((CH, H), jnp.float32) for _ in range(4)]
            + [pltpu.VMEM_SHARED((n, H), jnp.float32)]
            + [pltpu.SemaphoreType.DMA for _ in range(8)]
        ),
    )
    def scatter(*refs):
        ws = refs[:nslab]
        (dst2_hbm, zeros_hbm, out_hbm, idxg, b0, b1, b2, b3,
         acc, d0, d1, d2, d3, a0, a1, a2, a3) = refs[nslab:]
        core = lax.axis_index("c")
        sid = lax.axis_index("s")
        bufs = (b0, b1, b2, b3)
        dsems = (d0, d1, d2, d3)
        asems = (a0, a1, a2, a3)
        for p in range(2):
            plane = core * 2 + p

            @pl.when(sid < nwb)
            def _():
                pltpu.sync_copy(zeros_hbm, acc.at[pl.ds(sid * nrow, nrow)])

            plsc.subcore_barrier()

            for s in range(nslab):
                w4_hbm = ws[s]

                @pl.loop(0, iters)
                def _(i):
                    unit = i * NS + sid

                    @pl.when(unit < nunits)
                    def _():
                        row0 = unit * GRP
                        pltpu.sync_copy(
                            dst2_hbm.at[pl.ds(s * srows + row0, GRP)], idxg)
                        dh = [None] * GRP
                        ah = [None] * GRP
                        for k in range(GRP + 1):
                            if k < GRP:
                                b = k % 4
                                if k >= 4:
                                    ah[k - 4].wait()
                                dh[k] = pltpu.async_copy(
                                    w4_hbm.at[plane,
                                              pl.ds((row0 + k) * CH, CH)],
                                    bufs[b], dsems[b])
                            if k >= 1:
                                j = k - 1
                                dh[j].wait()
                                ah[j] = pltpu.async_copy(
                                    bufs[j % 4], acc.at[idxg.at[j]],
                                    asems[j % 4], add=True)
                        for j in range(GRP - 4, GRP):
                            ah[j].wait()

            plsc.subcore_barrier()

            @pl.when(sid < nwb)
            def _():
                pltpu.sync_copy(
                    acc.at[pl.ds(sid * nrow, nrow)],
                    out_hbm.at[plane, pl.ds(sid * nrow, nrow)])

            plsc.subcore_barrier()

    return scatter


def kernel(s_j, v_j, edge_index, edge_rbf, edge_vec, W1, b1, W2, b2, Wr, br):
    n = s_j.shape[0]
    e = edge_index.shape[1]
    vj2 = v_j.reshape(n, H3)
    src2 = edge_index[0].reshape(e // CH, CH)
    dst2 = edge_index[1].reshape(e // CH, CH)

    nslab = 4
    es = e // nslab
    srows = es // CH
    tab = _node_table(s_j, vj2, W1, b1, W2, b2)
    gather_fn = _make_gather(n, es)
    w4s = []
    for s in range(nslab):
        g_s = gather_fn(tab, src2[s * srows:(s + 1) * srows])
        w4s.append(_edge_math(g_s, edge_rbf[s * es:(s + 1) * es],
                              edge_vec[s * es:(s + 1) * es], Wr, br))
    zeros = jnp.zeros((n // 10, H), jnp.float32)
    out4 = _make_scatter(n, e, nslab)(*w4s, dst2, zeros)

    delta_s = out4[0]
    delta_v = jnp.transpose(out4[1:4], (1, 0, 2))
    return (delta_s, delta_v)


# speedup vs baseline: 1.0623x; 1.0623x over previous
import functools
import math

import jax
import jax.numpy as jnp
from jax import lax
from jax.experimental import pallas as pl
from jax.experimental.pallas import tpu as pltpu
from jax.experimental.pallas import tpu_sc as plsc

H = 128
H3 = 3 * H
TW = 2 * H3
TWP = TW // 2
HP = H3 // 2
NRAD = 20

NC = 2
NS = 16
NW = NC * NS

CH = 80
GRP = 16


def _bf16_rnd(x):
    b = lax.bitcast_convert_type(x, jnp.uint32)
    return b + jnp.uint32(0x7FFF) + (
        lax.shift_right_logical(b, jnp.uint32(16)) & jnp.uint32(1))


def _pack_pair(lo_f32, hi_f32):
    lo = lax.shift_right_logical(_bf16_rnd(lo_f32), jnp.uint32(16))
    hi = _bf16_rnd(hi_f32) & jnp.uint32(0xFFFF0000)
    return lax.bitcast_convert_type(lo | hi, jnp.int32)


def _unpack_pair(p):
    u = lax.bitcast_convert_type(p, jnp.uint32)
    lo = lax.bitcast_convert_type(lax.shift_left(u, jnp.uint32(16)),
                                  jnp.float32)
    hi = lax.bitcast_convert_type(u & jnp.uint32(0xFFFF0000), jnp.float32)
    return lo, hi


def _mlp_body(s_ref, vj_ref, w1_ref, b1_ref, w2_ref, b2_ref, out_ref):
    s = s_ref[...]
    h = lax.dot_general(s, w1_ref[...], (((1,), (1,)), ((), ())),
                        preferred_element_type=jnp.float32) + b1_ref[...]
    h = h * jax.nn.sigmoid(h)
    mf = lax.dot_general(h, w2_ref[...], (((1,), (1,)), ((), ())),
                         preferred_element_type=jnp.float32) + b2_ref[...]
    vj = vj_ref[...]
    out_ref[:, :H] = _pack_pair(mf[:, :H], mf[:, H:2 * H])
    out_ref[:, H:2 * H] = _pack_pair(mf[:, 2 * H:], vj[:, :H])
    out_ref[:, 2 * H:] = _pack_pair(vj[:, H:2 * H], vj[:, 2 * H:])


def _node_table(s_j, vj2, W1, b1, W2, b2):
    n = s_j.shape[0]
    bn = 2000
    return pl.pallas_call(
        _mlp_body,
        grid=(n // bn,),
        in_specs=[
            pl.BlockSpec((bn, H), lambda i: (i, 0)),
            pl.BlockSpec((bn, H3), lambda i: (i, 0)),
            pl.BlockSpec((H, H), lambda i: (0, 0)),
            pl.BlockSpec((1, H), lambda i: (0, 0)),
            pl.BlockSpec((H3, H), lambda i: (0, 0)),
            pl.BlockSpec((1, H3), lambda i: (0, 0)),
        ],
        out_specs=pl.BlockSpec((bn, TWP), lambda i: (i, 0)),
        out_shape=jax.ShapeDtypeStruct((n, TWP), jnp.int32),
        compiler_params=pltpu.CompilerParams(
            dimension_semantics=("parallel",)),
    )(s_j, vj2, W1, b1.reshape(1, H), W2, b2.reshape(1, H3))


def _make_gather(n, e):
    nunits = e // (CH * GRP)
    iters = -(-nunits // NW)
    mesh = plsc.VectorSubcoreMesh(core_axis_name="c", subcore_axis_name="s")

    @functools.partial(
        pl.kernel,
        out_type=jax.ShapeDtypeStruct((e, TWP), jnp.int32),
        mesh=mesh,
        scratch_types=(
            [pltpu.VMEM((GRP, CH), jnp.int32)]
            + [pltpu.VMEM((CH, TWP), jnp.int32) for _ in range(4)]
            + [pltpu.SemaphoreType.DMA for _ in range(8)]
        ),
    )
    def gather(tab_hbm, src2_hbm, g_hbm, idxg, b0, b1, b2, b3,
               g0, g1, g2, g3, w0, w1, w2, w3):
        wid = lax.axis_index("s") * NC + lax.axis_index("c")
        bufs = (b0, b1, b2, b3)
        gsems = (g0, g1, g2, g3)
        wsems = (w0, w1, w2, w3)

        @pl.loop(0, iters)
        def _(i):
            unit = i * NW + wid

            @pl.when(unit < nunits)
            def _():
                row0 = unit * GRP
                pltpu.sync_copy(src2_hbm.at[pl.ds(row0, GRP)], idxg)
                gh = [None] * GRP
                wh = [None] * GRP
                for k in range(GRP + 1):
                    if k < GRP:
                        b = k % 4
                        if k >= 4:
                            wh[k - 4].wait()
                        gh[k] = pltpu.async_copy(
                            tab_hbm.at[idxg.at[k]], bufs[b], gsems[b])
                    if k >= 1:
                        j = k - 1
                        gh[j].wait()
                        wh[j] = pltpu.async_copy(
                            bufs[j % 4],
                            g_hbm.at[pl.ds((row0 + j) * CH, CH)],
                            wsems[j % 4])
                for j in range(GRP - 4, GRP):
                    wh[j].wait()

    return gather


_S_VS = 1.0 / (math.sqrt(3.0) * math.sqrt(float(H)))
_S_VV = 1.0 / math.sqrt(float(H))


def _edge_body(g_ref, rbf_ref, vec_ref, wr_ref, br_ref, w_ref):
    wf = lax.dot_general(rbf_ref[...], wr_ref[...], (((1,), (1,)), ((), ())),
                         preferred_element_type=jnp.float32) + br_ref[...]
    gi = g_ref[...]
    mf0, mf1 = _unpack_pair(gi[:, :H])
    mf2, vj0 = _unpack_pair(gi[:, H:2 * H])
    vj1, vj2 = _unpack_pair(gi[:, 2 * H:])
    w_ref[0] = mf0 * wf[:, :H]
    w_vs = mf1 * wf[:, H:2 * H] * _S_VS
    w_vv = mf2 * wf[:, 2 * H:] * _S_VV
    vec = vec_ref[...]
    for c, vjc in enumerate((vj0, vj1, vj2)):
        w_ref[c + 1] = vjc * w_vs + w_vv * vec[:, c:c + 1]


def _edge_math(g, edge_rbf, edge_vec, Wr, br):
    e = g.shape[0]
    be = 4000
    return pl.pallas_call(
        _edge_body,
        grid=(e // be,),
        in_specs=[
            pl.BlockSpec((be, TWP), lambda i: (i, 0)),
            pl.BlockSpec((be, NRAD), lambda i: (i, 0)),
            pl.BlockSpec((be, 3), lambda i: (i, 0)),
            pl.BlockSpec((H3, NRAD), lambda i: (0, 0)),
            pl.BlockSpec((1, H3), lambda i: (0, 0)),
        ],
        out_specs=pl.BlockSpec((4, be, H), lambda i: (0, i, 0)),
        out_shape=jax.ShapeDtypeStruct((4, e, H), jnp.float32),
        compiler_params=pltpu.CompilerParams(
            dimension_semantics=("parallel",)),
    )(g, edge_rbf, edge_vec, Wr, br.reshape(1, H3))


def _make_scatter(n, e, nslab):
    es = e // nslab
    srows = es // CH
    nunits = es // (CH * GRP)
    iters = -(-nunits // NS)
    nwb = 10
    nrow = n // nwb
    mesh = plsc.VectorSubcoreMesh(core_axis_name="c", subcore_axis_name="s")

    @functools.partial(
        pl.kernel,
        out_type=jax.ShapeDtypeStruct((4, n, H), jnp.float32),
        mesh=mesh,
        scratch_types=(
            [pltpu.VMEM((GRP, CH), jnp.int32)]
            + [pltpu.VMEM((CH, H), jnp.float32) for _ in range(4)]
            + [pltpu.VMEM_SHARED((n, H), jnp.float32)]
            + [pltpu.SemaphoreType.DMA for _ in range(8)]
        ),
    )
    def scatter(*refs):
        ws = refs[:nslab]
        (dst2_hbm, zeros_hbm, out_hbm, idxg, b0, b1, b2, b3,
         acc, d0, d1, d2, d3, a0, a1, a2, a3) = refs[nslab:]
        core = lax.axis_index("c")
        sid = lax.axis_index("s")
        bufs = (b0, b1, b2, b3)
        dsems = (d0, d1, d2, d3)
        asems = (a0, a1, a2, a3)
        for p in range(2):
            plane = core * 2 + p

            @pl.when(sid < nwb)
            def _():
                pltpu.sync_copy(zeros_hbm, acc.at[pl.ds(sid * nrow, nrow)])

            plsc.subcore_barrier()

            for s in range(nslab):
                w4_hbm = ws[s]

                @pl.loop(0, iters)
                def _(i):
                    unit = i * NS + sid

                    @pl.when(unit < nunits)
                    def _():
                        row0 = unit * GRP
                        pltpu.sync_copy(
                            dst2_hbm.at[pl.ds(s * srows + row0, GRP)], idxg)
                        dh = [None] * GRP
                        ah = [None] * GRP
                        for k in range(GRP + 1):
                            if k < GRP:
                                b = k % 4
                                if k >= 4:
                                    ah[k - 4].wait()
                                dh[k] = pltpu.async_copy(
                                    w4_hbm.at[plane,
                                              pl.ds((row0 + k) * CH, CH)],
                                    bufs[b], dsems[b])
                            if k >= 1:
                                j = k - 1
                                dh[j].wait()
                                ah[j] = pltpu.async_copy(
                                    bufs[j % 4], acc.at[idxg.at[j]],
                                    asems[j % 4], add=True)
                        for j in range(GRP - 4, GRP):
                            ah[j].wait()

            plsc.subcore_barrier()

            @pl.when(sid < nwb)
            def _():
                pltpu.sync_copy(
                    acc.at[pl.ds(sid * nrow, nrow)],
                    out_hbm.at[plane, pl.ds(sid * nrow, nrow)])

            plsc.subcore_barrier()

    return scatter


def kernel(s_j, v_j, edge_index, edge_rbf, edge_vec, W1, b1, W2, b2, Wr, br):
    n = s_j.shape[0]
    e = edge_index.shape[1]
    vj2 = v_j.reshape(n, H3)
    src2 = edge_index[0].reshape(e // CH, CH)
    dst2 = edge_index[1].reshape(e // CH, CH)

    nslab = 1
    es = e // nslab
    srows = es // CH
    tab = _node_table(s_j, vj2, W1, b1, W2, b2)
    gather_fn = _make_gather(n, es)
    w4s = []
    for s in range(nslab):
        g_s = gather_fn(tab, src2[s * srows:(s + 1) * srows])
        w4s.append(_edge_math(g_s, edge_rbf[s * es:(s + 1) * es],
                              edge_vec[s * es:(s + 1) * es], Wr, br))
    zeros = jnp.zeros((n // 10, H), jnp.float32)
    out4 = _make_scatter(n, e, nslab)(*w4s, dst2, zeros)

    delta_s = out4[0]
    delta_v = jnp.transpose(out4[1:4], (1, 0, 2))
    return (delta_s, delta_v)


# speedup vs baseline: 1.1884x; 1.1187x over previous
import functools
import math

import jax
import jax.numpy as jnp
from jax import lax
from jax.experimental import pallas as pl
from jax.experimental.pallas import tpu as pltpu
from jax.experimental.pallas import tpu_sc as plsc

H = 128
H3 = 3 * H
TW = 2 * H3
TWP = TW // 2
HP = H3 // 2
NRAD = 20

NC = 2
NS = 16
NW = NC * NS

CH = 80
GRP = 16


def _bf16_rnd(x):
    b = lax.bitcast_convert_type(x, jnp.uint32)
    return b + jnp.uint32(0x7FFF) + (
        lax.shift_right_logical(b, jnp.uint32(16)) & jnp.uint32(1))


def _pack_pair(lo_f32, hi_f32):
    lo = lax.shift_right_logical(_bf16_rnd(lo_f32), jnp.uint32(16))
    hi = _bf16_rnd(hi_f32) & jnp.uint32(0xFFFF0000)
    return lax.bitcast_convert_type(lo | hi, jnp.int32)


def _unpack_pair(p):
    u = lax.bitcast_convert_type(p, jnp.uint32)
    lo = lax.bitcast_convert_type(lax.shift_left(u, jnp.uint32(16)),
                                  jnp.float32)
    hi = lax.bitcast_convert_type(u & jnp.uint32(0xFFFF0000), jnp.float32)
    return lo, hi


def _mlp_body(s_ref, vj_ref, w1_ref, b1_ref, w2_ref, b2_ref, out_ref):
    s = s_ref[...]
    h = lax.dot_general(s, w1_ref[...], (((1,), (1,)), ((), ())),
                        preferred_element_type=jnp.float32) + b1_ref[...]
    h = h * jax.nn.sigmoid(h)
    mf = lax.dot_general(h, w2_ref[...], (((1,), (1,)), ((), ())),
                         preferred_element_type=jnp.float32) + b2_ref[...]
    out_ref[:, :H] = _pack_pair(mf[:, :H], mf[:, H:2 * H])
    out_ref[:, H:2 * H] = _pack_pair(mf[:, 2 * H:], vj_ref[:, 0, :])
    out_ref[:, 2 * H:] = _pack_pair(vj_ref[:, 1, :], vj_ref[:, 2, :])


def _node_table(s_j, v_j, W1, b1, W2, b2):
    n = s_j.shape[0]
    bn = 2000
    return pl.pallas_call(
        _mlp_body,
        grid=(n // bn,),
        in_specs=[
            pl.BlockSpec((bn, H), lambda i: (i, 0)),
            pl.BlockSpec((bn, 3, H), lambda i: (i, 0, 0)),
            pl.BlockSpec((H, H), lambda i: (0, 0)),
            pl.BlockSpec((1, H), lambda i: (0, 0)),
            pl.BlockSpec((H3, H), lambda i: (0, 0)),
            pl.BlockSpec((1, H3), lambda i: (0, 0)),
        ],
        out_specs=pl.BlockSpec((bn, TWP), lambda i: (i, 0)),
        out_shape=jax.ShapeDtypeStruct((n, TWP), jnp.int32),
        compiler_params=pltpu.CompilerParams(
            dimension_semantics=("parallel",)),
    )(s_j, v_j, W1, b1.reshape(1, H), W2, b2.reshape(1, H3))


def _make_gather(n, es, srow0):
    nunits = es // (CH * GRP)
    iters = -(-nunits // NW)
    mesh = plsc.VectorSubcoreMesh(core_axis_name="c", subcore_axis_name="s")

    @functools.partial(
        pl.kernel,
        out_type=jax.ShapeDtypeStruct((es, TWP), jnp.int32),
        mesh=mesh,
        scratch_types=(
            [pltpu.VMEM((GRP, CH), jnp.int32)]
            + [pltpu.VMEM((CH, TWP), jnp.int32) for _ in range(4)]
            + [pltpu.SemaphoreType.DMA for _ in range(8)]
        ),
    )
    def gather(tab_hbm, src2_hbm, g_hbm, idxg, b0, b1, b2, b3,
               g0, g1, g2, g3, w0, w1, w2, w3):
        wid = lax.axis_index("s") * NC + lax.axis_index("c")
        bufs = (b0, b1, b2, b3)
        gsems = (g0, g1, g2, g3)
        wsems = (w0, w1, w2, w3)

        @pl.loop(0, iters)
        def _(i):
            unit = i * NW + wid

            @pl.when(unit < nunits)
            def _():
                row0 = unit * GRP
                pltpu.sync_copy(src2_hbm.at[pl.ds(srow0 + row0, GRP)], idxg)
                gh = [None] * GRP
                wh = [None] * GRP
                for k in range(GRP + 1):
                    if k < GRP:
                        b = k % 4
                        if k >= 4:
                            wh[k - 4].wait()
                        gh[k] = pltpu.async_copy(
                            tab_hbm.at[idxg.at[k]], bufs[b], gsems[b])
                    if k >= 1:
                        j = k - 1
                        gh[j].wait()
                        wh[j] = pltpu.async_copy(
                            bufs[j % 4],
                            g_hbm.at[pl.ds((row0 + j) * CH, CH)],
                            wsems[j % 4])
                for j in range(GRP - 4, GRP):
                    wh[j].wait()

    return gather


_S_VS = 1.0 / (math.sqrt(3.0) * math.sqrt(float(H)))
_S_VV = 1.0 / math.sqrt(float(H))


def _edge_body(g_ref, rbft_ref, vect_ref, wr_ref, br_ref, sel_ref, w_ref):
    wf = lax.dot_general(rbft_ref[...], wr_ref[...], (((0,), (1,)), ((), ())),
                         preferred_element_type=jnp.float32) + br_ref[...]
    vecb = lax.dot_general(vect_ref[...], sel_ref[...], (((0,), (0,)), ((), ())),
                           preferred_element_type=jnp.float32)
    gi = g_ref[...]
    mf0, mf1 = _unpack_pair(gi[:, :H])
    mf2, vj0 = _unpack_pair(gi[:, H:2 * H])
    vj1, vj2 = _unpack_pair(gi[:, 2 * H:])
    w_ref[0] = mf0 * wf[:, :H]
    w_vs = mf1 * wf[:, H:2 * H] * _S_VS
    w_vv = mf2 * wf[:, 2 * H:] * _S_VV
    for c, vjc in enumerate((vj0, vj1, vj2)):
        w_ref[c + 1] = vjc * w_vs + w_vv * vecb[:, c * H:(c + 1) * H]


def _edge_math(rbf_t, vec_t, Wr, br, sel, slab, es):
    be = 3200
    b0 = slab * (es // be)

    def call(g):
        return pl.pallas_call(
            _edge_body,
            grid=(es // be,),
            in_specs=[
                pl.BlockSpec((be, TWP), lambda i: (i, 0)),
                pl.BlockSpec((NRAD, be), lambda i: (0, b0 + i)),
                pl.BlockSpec((3, be), lambda i: (0, b0 + i)),
                pl.BlockSpec((H3, NRAD), lambda i: (0, 0)),
                pl.BlockSpec((1, H3), lambda i: (0, 0)),
                pl.BlockSpec((3, H3), lambda i: (0, 0)),
            ],
            out_specs=pl.BlockSpec((4, be, H), lambda i: (0, i, 0)),
            out_shape=jax.ShapeDtypeStruct((4, es, H), jnp.float32),
            compiler_params=pltpu.CompilerParams(
                dimension_semantics=("parallel",)),
        )(g, rbf_t, vec_t, Wr, br.reshape(1, H3), sel)

    return call


def _make_scatter(n, e, nslab):
    es = e // nslab
    srows = es // CH
    nunits = es // (CH * GRP)
    iters = -(-nunits // NS)
    nwb = 10
    nrow = n // nwb
    mesh = plsc.VectorSubcoreMesh(core_axis_name="c", subcore_axis_name="s")

    @functools.partial(
        pl.kernel,
        out_type=jax.ShapeDtypeStruct((4, n, H), jnp.float32),
        mesh=mesh,
        scratch_types=(
            [pltpu.VMEM((GRP, CH), jnp.int32)]
            + [pltpu.VMEM((CH, H), jnp.float32) for _ in range(4)]
            + [pltpu.VMEM_SHARED((n, H), jnp.float32)]
            + [pltpu.SemaphoreType.DMA for _ in range(8)]
        ),
    )
    def scatter(*refs):
        ws = refs[:nslab]
        (dst2_hbm, zeros_hbm, out_hbm, idxg, b0, b1, b2, b3,
         acc, d0, d1, d2, d3, a0, a1, a2, a3) = refs[nslab:]
        core = lax.axis_index("c")
        sid = lax.axis_index("s")
        bufs = (b0, b1, b2, b3)
        dsems = (d0, d1, d2, d3)
        asems = (a0, a1, a2, a3)
        for p in range(2):
            plane = core * 2 + p

            @pl.when(sid < nwb)
            def _():
                pltpu.sync_copy(zeros_hbm, acc.at[pl.ds(sid * nrow, nrow)])

            plsc.subcore_barrier()

            for s in range(nslab):
                w4_hbm = ws[s]

                @pl.loop(0, iters)
                def _(i):
                    unit = i * NS + sid

                    @pl.when(unit < nunits)
                    def _():
                        row0 = unit * GRP
                        pltpu.sync_copy(
                            dst2_hbm.at[pl.ds(s * srows + row0, GRP)], idxg)
                        dh = [None] * GRP
                        ah = [None] * GRP
                        for k in range(GRP + 1):
                            if k < GRP:
                                b = k % 4
                                if k >= 4:
                                    ah[k - 4].wait()
                                dh[k] = pltpu.async_copy(
                                    w4_hbm.at[plane,
                                              pl.ds((row0 + k) * CH, CH)],
                                    bufs[b], dsems[b])
                            if k >= 1:
                                j = k - 1
                                dh[j].wait()
                                ah[j] = pltpu.async_copy(
                                    bufs[j % 4], acc.at[idxg.at[j]],
                                    asems[j % 4], add=True)
                        for j in range(GRP - 4, GRP):
                            ah[j].wait()

            plsc.subcore_barrier()

            @pl.when(sid < nwb)
            def _():
                pltpu.sync_copy(
                    acc.at[pl.ds(sid * nrow, nrow)],
                    out_hbm.at[plane, pl.ds(sid * nrow, nrow)])

            plsc.subcore_barrier()

    return scatter


def kernel(s_j, v_j, edge_index, edge_rbf, edge_vec, W1, b1, W2, b2, Wr, br):
    n = s_j.shape[0]
    e = edge_index.shape[1]
    src2 = edge_index[0].reshape(e // CH, CH)
    dst2 = edge_index[1].reshape(e // CH, CH)
    rbf_t = edge_rbf.T
    vec_t = edge_vec.T
    sel = jnp.zeros((3, H3), jnp.float32)
    for c in range(3):
        sel = sel.at[c, c * H:(c + 1) * H].set(1.0)

    nslab = 2
    es = e // nslab
    srows = es // CH
    tab = _node_table(s_j, v_j, W1, b1, W2, b2)
    w4s = []
    for s in range(nslab):
        g_s = _make_gather(n, es, s * srows)(tab, src2)
        w4s.append(_edge_math(rbf_t, vec_t, Wr, br, sel, s, es)(g_s))
    zeros = jnp.zeros((n // 10, H), jnp.float32)
    out4 = _make_scatter(n, e, nslab)(*w4s, dst2, zeros)

    delta_s = out4[0]
    delta_v = jnp.transpose(out4[1:4], (1, 0, 2))
    return (delta_s, delta_v)


# speedup vs baseline: 1.3022x; 1.0958x over previous
import functools
import math

import jax
import jax.numpy as jnp
from jax import lax
from jax.experimental import pallas as pl
from jax.experimental.pallas import tpu as pltpu
from jax.experimental.pallas import tpu_sc as plsc

H = 128
H3 = 3 * H
TW = 2 * H3
TWP = TW // 2
HP = H3 // 2
NRAD = 20

NC = 2
NS = 16
NW = NC * NS

CH = 80
GRP = 16


def _bf16_rnd(x):
    b = lax.bitcast_convert_type(x, jnp.uint32)
    return b + jnp.uint32(0x7FFF) + (
        lax.shift_right_logical(b, jnp.uint32(16)) & jnp.uint32(1))


def _pack_pair(lo_f32, hi_f32):
    lo = lax.shift_right_logical(_bf16_rnd(lo_f32), jnp.uint32(16))
    hi = _bf16_rnd(hi_f32) & jnp.uint32(0xFFFF0000)
    return lax.bitcast_convert_type(lo | hi, jnp.int32)


def _unpack_pair(p):
    u = lax.bitcast_convert_type(p, jnp.uint32)
    lo = lax.bitcast_convert_type(lax.shift_left(u, jnp.uint32(16)),
                                  jnp.float32)
    hi = lax.bitcast_convert_type(u & jnp.uint32(0xFFFF0000), jnp.float32)
    return lo, hi


def _mlp_body(s_ref, vj_ref, w1_ref, b1_ref, w2_ref, b2_ref, out_ref):
    s = s_ref[...]
    h = lax.dot_general(s, w1_ref[...], (((1,), (1,)), ((), ())),
                        preferred_element_type=jnp.float32) + b1_ref[...]
    h = h * jax.nn.sigmoid(h)
    mf = lax.dot_general(h, w2_ref[...], (((1,), (1,)), ((), ())),
                         preferred_element_type=jnp.float32) + b2_ref[...]
    out_ref[:, :H] = _pack_pair(mf[:, :H], mf[:, H:2 * H])
    out_ref[:, H:2 * H] = _pack_pair(mf[:, 2 * H:], vj_ref[0])
    out_ref[:, 2 * H:] = _pack_pair(vj_ref[1], vj_ref[2])


def _node_table(s_j, v_j, W1, b1, W2, b2):
    n = s_j.shape[0]
    bn = 2000
    return pl.pallas_call(
        _mlp_body,
        grid=(n // bn,),
        in_specs=[
            pl.BlockSpec((bn, H), lambda i: (i, 0)),
            pl.BlockSpec((3, bn, H), lambda i: (0, i, 0)),
            pl.BlockSpec((H, H), lambda i: (0, 0)),
            pl.BlockSpec((1, H), lambda i: (0, 0)),
            pl.BlockSpec((H3, H), lambda i: (0, 0)),
            pl.BlockSpec((1, H3), lambda i: (0, 0)),
        ],
        out_specs=pl.BlockSpec((bn, TWP), lambda i: (i, 0)),
        out_shape=jax.ShapeDtypeStruct((n, TWP), jnp.int32),
        compiler_params=pltpu.CompilerParams(
            dimension_semantics=("parallel",)),
    )(s_j, v_j.transpose(1, 0, 2), W1, b1.reshape(1, H), W2,
      b2.reshape(1, H3))


def _make_gather(n, es, srow0):
    nunits = es // (CH * GRP)
    iters = -(-nunits // NW)
    mesh = plsc.VectorSubcoreMesh(core_axis_name="c", subcore_axis_name="s")

    @functools.partial(
        pl.kernel,
        out_type=jax.ShapeDtypeStruct((es, TWP), jnp.int32),
        mesh=mesh,
        scratch_types=(
            [pltpu.VMEM((GRP, CH), jnp.int32)]
            + [pltpu.VMEM((CH, TWP), jnp.int32) for _ in range(4)]
            + [pltpu.SemaphoreType.DMA for _ in range(8)]
        ),
    )
    def gather(tab_hbm, src2_hbm, g_hbm, idxg, b0, b1, b2, b3,
               g0, g1, g2, g3, w0, w1, w2, w3):
        wid = lax.axis_index("s") * NC + lax.axis_index("c")
        bufs = (b0, b1, b2, b3)
        gsems = (g0, g1, g2, g3)
        wsems = (w0, w1, w2, w3)

        @pl.loop(0, iters)
        def _(i):
            unit = i * NW + wid

            @pl.when(unit < nunits)
            def _():
                row0 = unit * GRP
                pltpu.sync_copy(src2_hbm.at[pl.ds(srow0 + row0, GRP)], idxg)
                gh = [None] * GRP
                wh = [None] * GRP
                for k in range(GRP + 1):
                    if k < GRP:
                        b = k % 4
                        if k >= 4:
                            wh[k - 4].wait()
                        gh[k] = pltpu.async_copy(
                            tab_hbm.at[idxg.at[k]], bufs[b], gsems[b])
                    if k >= 1:
                        j = k - 1
                        gh[j].wait()
                        wh[j] = pltpu.async_copy(
                            bufs[j % 4],
                            g_hbm.at[pl.ds((row0 + j) * CH, CH)],
                            wsems[j % 4])
                for j in range(GRP - 4, GRP):
                    wh[j].wait()

    return gather


_S_VS = 1.0 / (math.sqrt(3.0) * math.sqrt(float(H)))
_S_VV = 1.0 / math.sqrt(float(H))


def _edge_body(g_ref, rbft_ref, vect_ref, wr_ref, br_ref, sel_ref, w_ref):
    wf = lax.dot_general(rbft_ref[...], wr_ref[...], (((0,), (1,)), ((), ())),
                         preferred_element_type=jnp.float32) + br_ref[...]
    vecb = lax.dot_general(vect_ref[...], sel_ref[...], (((0,), (0,)), ((), ())),
                           preferred_element_type=jnp.float32)
    gi = g_ref[...]
    mf0, mf1 = _unpack_pair(gi[:, :H])
    mf2, vj0 = _unpack_pair(gi[:, H:2 * H])
    vj1, vj2 = _unpack_pair(gi[:, 2 * H:])
    w_ref[0] = mf0 * wf[:, :H]
    w_vs = mf1 * wf[:, H:2 * H] * _S_VS
    w_vv = mf2 * wf[:, 2 * H:] * _S_VV
    for c, vjc in enumerate((vj0, vj1, vj2)):
        w_ref[c + 1] = vjc * w_vs + w_vv * vecb[:, c * H:(c + 1) * H]


def _edge_math(rbf_t, vec_t, Wr, br, sel, slab, es):
    be = 3200
    b0 = slab * (es // be)

    def call(g):
        return pl.pallas_call(
            _edge_body,
            grid=(es // be,),
            in_specs=[
                pl.BlockSpec((be, TWP), lambda i: (i, 0)),
                pl.BlockSpec((NRAD, be), lambda i: (0, b0 + i)),
                pl.BlockSpec((3, be), lambda i: (0, b0 + i)),
                pl.BlockSpec((H3, NRAD), lambda i: (0, 0)),
                pl.BlockSpec((1, H3), lambda i: (0, 0)),
                pl.BlockSpec((3, H3), lambda i: (0, 0)),
            ],
            out_specs=pl.BlockSpec((4, be, H), lambda i: (0, i, 0)),
            out_shape=jax.ShapeDtypeStruct((4, es, H), jnp.float32),
            compiler_params=pltpu.CompilerParams(
                dimension_semantics=("parallel",)),
        )(g, rbf_t, vec_t, Wr, br.reshape(1, H3), sel)

    return call


def _make_scatter(n, es, srow0):
    nunits = es // (CH * GRP)
    iters = -(-nunits // NS)
    nwb = 10
    nrow = n // nwb
    mesh = plsc.VectorSubcoreMesh(core_axis_name="c", subcore_axis_name="s")

    @functools.partial(
        pl.kernel,
        out_type=jax.ShapeDtypeStruct((4, n, H), jnp.float32),
        mesh=mesh,
        scratch_types=(
            [pltpu.VMEM((GRP, CH), jnp.int32)]
            + [pltpu.VMEM((CH, H), jnp.float32) for _ in range(4)]
            + [pltpu.VMEM_SHARED((n, H), jnp.float32)]
            + [pltpu.SemaphoreType.DMA for _ in range(8)]
        ),
    )
    def scatter(w4_hbm, dst2_hbm, prev_hbm, out_hbm, idxg, b0, b1, b2, b3,
                acc, d0, d1, d2, d3, a0, a1, a2, a3):
        core = lax.axis_index("c")
        sid = lax.axis_index("s")
        bufs = (b0, b1, b2, b3)
        dsems = (d0, d1, d2, d3)
        asems = (a0, a1, a2, a3)
        for p in range(2):
            plane = core * 2 + p

            @pl.when(sid < nwb)
            def _():
                pltpu.sync_copy(prev_hbm.at[plane, pl.ds(sid * nrow, nrow)],
                                acc.at[pl.ds(sid * nrow, nrow)])

            plsc.subcore_barrier()

            @pl.loop(0, iters)
            def _(i):
                unit = i * NS + sid

                @pl.when(unit < nunits)
                def _():
                    row0 = unit * GRP
                    pltpu.sync_copy(
                        dst2_hbm.at[pl.ds(srow0 + row0, GRP)], idxg)
                    dh = [None] * GRP
                    ah = [None] * GRP
                    for k in range(GRP + 1):
                        if k < GRP:
                            b = k % 4
                            if k >= 4:
                                ah[k - 4].wait()
                            dh[k] = pltpu.async_copy(
                                w4_hbm.at[plane,
                                          pl.ds((row0 + k) * CH, CH)],
                                bufs[b], dsems[b])
                        if k >= 1:
                            j = k - 1
                            dh[j].wait()
                            ah[j] = pltpu.async_copy(
                                bufs[j % 4], acc.at[idxg.at[j]],
                                asems[j % 4], add=True)
                    for j in range(GRP - 4, GRP):
                        ah[j].wait()

            plsc.subcore_barrier()

            @pl.when(sid < nwb)
            def _():
                pltpu.sync_copy(
                    acc.at[pl.ds(sid * nrow, nrow)],
                    out_hbm.at[plane, pl.ds(sid * nrow, nrow)])

            plsc.subcore_barrier()

    return scatter


def kernel(s_j, v_j, edge_index, edge_rbf, edge_vec, W1, b1, W2, b2, Wr, br):
    n = s_j.shape[0]
    e = edge_index.shape[1]
    src2 = edge_index[0].reshape(e // CH, CH)
    dst2 = edge_index[1].reshape(e // CH, CH)
    rbf_t = edge_rbf.T
    vec_t = edge_vec.T
    sel = jnp.zeros((3, H3), jnp.float32)
    for c in range(3):
        sel = sel.at[c, c * H:(c + 1) * H].set(1.0)

    nslab = 2
    es = e // nslab
    srows = es // CH
    tab = _node_table(s_j, v_j, W1, b1, W2, b2)
    w4s = []
    for s in range(nslab):
        g_s = _make_gather(n, es, s * srows)(tab, src2)
        w4s.append(_edge_math(rbf_t, vec_t, Wr, br, sel, s, es)(g_s))
    out4 = jnp.zeros((4, n, H), jnp.float32)
    for s in range(nslab):
        out4 = _make_scatter(n, es, s * srows)(w4s[s], dst2, out4)

    delta_s = out4[0]
    delta_v = jnp.transpose(out4[1:4], (1, 0, 2))
    return (delta_s, delta_v)
